# baseline (speedup 1.0000x reference)
import jax
import jax.numpy as jnp
from jax.experimental import pallas as pl
from jax.experimental.pallas import tpu as pltpu

M = 65536
B = 256
MAXCH = 8192
SLOTS = 6
LOOKAHEAD = 4

_CHUNK_ROWS = [1024, 1024, 2048, 4096] + [8192] * 7
assert sum(_CHUNK_ROWS) == M

_TASKS = []
_off = 0
for _rows in _CHUNK_ROWS:
    _TASKS.append((0, _off, _rows))
    _TASKS.append((1, _off, _rows))
    _off += _rows
_NTASK = len(_TASKS)


def _copy_body(i0_ref, i1_ref, o0_ref, o1_ref, buf, load_sems, store_sems):
    srcs = (i0_ref, i1_ref)
    dsts = (o0_ref, o1_ref)

    def _load_copy(t):
        a, off, rows = _TASKS[t]
        s = t % SLOTS
        return pltpu.make_async_copy(
            srcs[a].at[pl.ds(off, rows), :],
            buf.at[s, pl.ds(0, rows), :],
            load_sems.at[s],
        )

    def _store_copy(t):
        a, off, rows = _TASKS[t]
        s = t % SLOTS
        return pltpu.make_async_copy(
            buf.at[s, pl.ds(0, rows), :],
            dsts[a].at[pl.ds(off, rows), :],
            store_sems.at[s],
        )

    for t in range(LOOKAHEAD):
        _load_copy(t).start()
    for t in range(_NTASK):
        _load_copy(t).wait()
        _store_copy(t).start()
        u = t + LOOKAHEAD
        if u < _NTASK:
            if u >= SLOTS:
                _store_copy(u - SLOTS).wait()
            _load_copy(u).start()
    for t in range(_NTASK - SLOTS, _NTASK):
        _store_copy(t).wait()


def kernel(input0, input1, input2, input3):
    del input2, input3
    anyspec = pl.BlockSpec(memory_space=pl.ANY)
    out0, out1 = pl.pallas_call(
        _copy_body,
        in_specs=[anyspec, anyspec],
        out_specs=[anyspec, anyspec],
        out_shape=[
            jax.ShapeDtypeStruct((M, B), jnp.float32),
            jax.ShapeDtypeStruct((M, B), jnp.float32),
        ],
        scratch_shapes=[
            pltpu.VMEM((SLOTS, MAXCH, B), jnp.float32),
            pltpu.SemaphoreType.DMA((SLOTS,)),
            pltpu.SemaphoreType.DMA((SLOTS,)),
        ],
    )(input0, input1)
    return (out0, out1)


# speedup vs baseline: 1.0025x; 1.0025x over previous
import jax
import jax.numpy as jnp
from jax.experimental import pallas as pl
from jax.experimental.pallas import tpu as pltpu

M = 65536
B = 256
MAXCH = 8192
SLOTS = 6
LOOKAHEAD = 5

_CHUNK_ROWS = [8192] * 8
assert sum(_CHUNK_ROWS) == M

_TASKS = []
_off = 0
for _rows in _CHUNK_ROWS:
    _TASKS.append((0, _off, _rows))
    _TASKS.append((1, _off, _rows))
    _off += _rows
_NTASK = len(_TASKS)


def _copy_body(i0_ref, i1_ref, o0_ref, o1_ref, buf, load_sems, store_sems):
    srcs = (i0_ref, i1_ref)
    dsts = (o0_ref, o1_ref)

    def _load_copy(t):
        a, off, rows = _TASKS[t]
        s = t % SLOTS
        return pltpu.make_async_copy(
            srcs[a].at[pl.ds(off, rows), :],
            buf.at[s, pl.ds(0, rows), :],
            load_sems.at[s],
        )

    def _store_copy(t):
        a, off, rows = _TASKS[t]
        s = t % SLOTS
        return pltpu.make_async_copy(
            buf.at[s, pl.ds(0, rows), :],
            dsts[a].at[pl.ds(off, rows), :],
            store_sems.at[s],
        )

    for t in range(LOOKAHEAD):
        _load_copy(t).start()
    for t in range(_NTASK):
        _load_copy(t).wait()
        _store_copy(t).start()
        u = t + LOOKAHEAD
        if u < _NTASK:
            if u >= SLOTS:
                _store_copy(u - SLOTS).wait()
            _load_copy(u).start()
    for t in range(_NTASK - SLOTS, _NTASK):
        _store_copy(t).wait()


def kernel(input0, input1, input2, input3):
    del input2, input3
    anyspec = pl.BlockSpec(memory_space=pl.ANY)
    out0, out1 = pl.pallas_call(
        _copy_body,
        in_specs=[anyspec, anyspec],
        out_specs=[anyspec, anyspec],
        out_shape=[
            jax.ShapeDtypeStruct((M, B), jnp.float32),
            jax.ShapeDtypeStruct((M, B), jnp.float32),
        ],
        scratch_shapes=[
            pltpu.VMEM((SLOTS, MAXCH, B), jnp.float32),
            pltpu.SemaphoreType.DMA((SLOTS,)),
            pltpu.SemaphoreType.DMA((SLOTS,)),
        ],
    )(input0, input1)
    return (out0, out1)


# speedup vs baseline: 1.0055x; 1.0030x over previous
import jax
import jax.numpy as jnp
from jax.experimental import pallas as pl
from jax.experimental.pallas import tpu as pltpu

M = 65536
B = 256
MAXCH = 8192
SLOTS = 7
LOOKAHEAD = 5

_CHUNK_ROWS = [8192] * 8
assert sum(_CHUNK_ROWS) == M

_TASKS = []
_off = 0
for _rows in _CHUNK_ROWS:
    _TASKS.append((0, _off, _rows))
    _TASKS.append((1, _off, _rows))
    _off += _rows
_NTASK = len(_TASKS)


def _copy_body(i0_ref, i1_ref, o0_ref, o1_ref, buf, load_sems, store_sems):
    srcs = (i0_ref, i1_ref)
    dsts = (o0_ref, o1_ref)

    def _load_copy(t):
        a, off, rows = _TASKS[t]
        s = t % SLOTS
        return pltpu.make_async_copy(
            srcs[a].at[pl.ds(off, rows), :],
            buf.at[s, pl.ds(0, rows), :],
            load_sems.at[s],
        )

    def _store_copy(t):
        a, off, rows = _TASKS[t]
        s = t % SLOTS
        return pltpu.make_async_copy(
            buf.at[s, pl.ds(0, rows), :],
            dsts[a].at[pl.ds(off, rows), :],
            store_sems.at[s],
        )

    for t in range(LOOKAHEAD):
        _load_copy(t).start()
    for t in range(_NTASK):
        _load_copy(t).wait()
        _store_copy(t).start()
        u = t + LOOKAHEAD
        if u < _NTASK:
            if u >= SLOTS:
                _store_copy(u - SLOTS).wait()
            _load_copy(u).start()
    for t in range(_NTASK - SLOTS, _NTASK):
        _store_copy(t).wait()


def kernel(input0, input1, input2, input3):
    del input2, input3
    anyspec = pl.BlockSpec(memory_space=pl.ANY)
    out0, out1 = pl.pallas_call(
        _copy_body,
        in_specs=[anyspec, anyspec],
        out_specs=[anyspec, anyspec],
        out_shape=[
            jax.ShapeDtypeStruct((M, B), jnp.float32),
            jax.ShapeDtypeStruct((M, B), jnp.float32),
        ],
        scratch_shapes=[
            pltpu.VMEM((SLOTS, MAXCH, B), jnp.float32),
            pltpu.SemaphoreType.DMA((SLOTS,)),
            pltpu.SemaphoreType.DMA((SLOTS,)),
        ],
    )(input0, input1)
    return (out0, out1)
